# Initial kernel scaffold; baseline (speedup 1.0000x reference)
import jax
import jax.numpy as jnp
from jax.experimental import pallas as pl


def kernel(x, edge_index, W1, b1, Wg0, bg0, Wg1, bg1, W2, b2):
    raise NotImplementedError("write your pallas kernel here")


# speedup vs baseline: 7.2141x; 7.2141x over previous
import functools

import jax
import jax.numpy as jnp
from jax import lax
from jax.experimental import pallas as pl
from jax.experimental.pallas import tpu as pltpu
from jax.experimental.pallas import tpu_sc as plsc

N = 10000
E = 320000
D = 128
EPS = 0.3

NC = 2
NS = 16
LANES = 16

NPAD = 10240
ROWS_PER_TILE = NPAD // NS
EPT = 10240
EPAD = NC * NS * EPT
BLK = 80
CPB = 8
NCHUNK = EPT // (BLK * CPB)


def _prep_body(x_ref, w1t_ref, b1_ref, g_ref, bgv_ref, deg_ref,
               h_ref, qr_ref, nrm_ref):
    h = jnp.maximum(jnp.dot(x_ref[...], w1t_ref[...],
                            preferred_element_type=jnp.float32)
                    + b1_ref[...], 0.0)
    h_ref[...] = h
    qr_ref[...] = jnp.dot(h, g_ref[...],
                          preferred_element_type=jnp.float32) + bgv_ref[...]
    deg = deg_ref[0] + deg_ref[1]
    nrm_ref[...] = 1.0 / jnp.sqrt(jnp.maximum(deg, 1.0))


def _mid_body(h_ref, agg_ref, g_ref, bgv_ref, h1_ref, qr_ref):
    h1 = EPS * h_ref[...] + agg_ref[0] + agg_ref[1]
    h1_ref[...] = h1
    qr_ref[...] = jnp.dot(h1, g_ref[...],
                          preferred_element_type=jnp.float32) + bgv_ref[...]


def _final_body(h_ref, agg_ref, w2t_ref, b2_ref, out_ref):
    h2 = EPS * h_ref[...] + agg_ref[0] + agg_ref[1]
    o = jnp.dot(h2, w2t_ref[...], preferred_element_type=jnp.float32) \
        + b2_ref[...]
    m = jnp.max(o, axis=1, keepdims=True)
    ls = jnp.log(jnp.sum(jnp.exp(o - m), axis=1, keepdims=True))
    out_ref[...] = o - m - ls


def _deg_body(dst4_hbm, out_hbm, zero_v, ones_v, idx_v, deg_sp):
    c = lax.axis_index("c")
    s = lax.axis_index("s")
    w = c * NS + s

    def _z(i, _):
        zero_v[pl.ds(i * LANES, LANES)] = jnp.zeros((LANES,), jnp.float32)
        return 0
    lax.fori_loop(0, ROWS_PER_TILE // LANES, _z, 0)
    for i in range(BLK // LANES):
        ones_v[pl.ds(i * LANES, LANES)] = jnp.ones((LANES,), jnp.float32)

    pltpu.sync_copy(zero_v, deg_sp.at[pl.ds(s * ROWS_PER_TILE,
                                            ROWS_PER_TILE)])
    pltpu.sync_copy(dst4_hbm.at[w], idx_v)
    plsc.subcore_barrier()

    def _blk(j, _):
        cc = j // CPB
        jj = j % CPB
        pltpu.sync_copy(ones_v, deg_sp.at[idx_v.at[cc, jj]], add=True)
        return 0
    lax.fori_loop(0, NCHUNK * CPB, _blk, 0)

    plsc.subcore_barrier()
    pltpu.sync_copy(deg_sp.at[pl.ds(s * ROWS_PER_TILE, ROWS_PER_TILE)],
                    out_hbm.at[c, pl.ds(s * ROWS_PER_TILE, ROWS_PER_TILE)])


def _fa_body(h_hbm, qd_hbm, rs_hbm, nrm_hbm, src4_hbm, dst4_hbm,
             out_hbm, qd_v, rs_v, nrm_v, src_v, dst_v, e_v, rows_v, sem,
             agg_sp):
    c = lax.axis_index("c")
    s = lax.axis_index("s")
    w = c * NS + s

    def _z(i, _):
        for k in range(D // LANES):
            rows_v[i, pl.ds(k * LANES, LANES)] = (
                jnp.zeros((LANES,), jnp.float32))
        return 0
    lax.fori_loop(0, BLK, _z, 0)
    for r in range(ROWS_PER_TILE // BLK):
        pltpu.sync_copy(
            rows_v, agg_sp.at[pl.ds(s * ROWS_PER_TILE + r * BLK, BLK)])

    pltpu.sync_copy(qd_hbm, qd_v)
    pltpu.sync_copy(rs_hbm, rs_v)
    pltpu.sync_copy(nrm_hbm, nrm_v)
    plsc.subcore_barrier()

    def _chunk(cc, _):
        pltpu.sync_copy(src4_hbm.at[w, cc], src_v)
        pltpu.sync_copy(dst4_hbm.at[w, cc], dst_v)

        def _blk(jj, _):
            gather = pltpu.async_copy(h_hbm.at[src_v.at[jj]], rows_v, sem)
            for k in range(BLK // LANES):
                sl = pl.ds(k * LANES, LANES)
                sidx = src_v[jj, sl]
                didx = dst_v[jj, sl]
                al = (plsc.load_gather(qd_v, [didx])
                      + plsc.load_gather(rs_v, [sidx]))
                t = 1.0 - 2.0 / (jnp.exp(2.0 * al) + 1.0)
                e_v[sl] = (t * plsc.load_gather(nrm_v, [sidx])
                           * plsc.load_gather(nrm_v, [didx]))
            gather.wait()

            def _scale(i, _):
                eb = plsc.load_gather(e_v, [lax.broadcast(i, (LANES,))])
                for k in range(D // LANES):
                    fl = pl.ds(k * LANES, LANES)
                    rows_v[i, fl] = rows_v[i, fl] * eb
                return 0
            lax.fori_loop(0, BLK, _scale, 0)

            pltpu.sync_copy(rows_v, agg_sp.at[dst_v.at[jj]], add=True)
            return 0
        lax.fori_loop(0, CPB, _blk, 0)
        return 0
    lax.fori_loop(0, NCHUNK, _chunk, 0)

    plsc.subcore_barrier()
    pltpu.sync_copy(agg_sp.at[pl.ds(s * ROWS_PER_TILE, ROWS_PER_TILE)],
                    out_hbm.at[c, pl.ds(s * ROWS_PER_TILE, ROWS_PER_TILE)])


_SC_MESH = plsc.VectorSubcoreMesh(core_axis_name="c", subcore_axis_name="s",
                                  num_cores=NC, num_subcores=NS)
_SC_PARAMS = pltpu.CompilerParams(needs_layout_passes=False)

_deg_kernel = pl.kernel(
    _deg_body,
    out_type=jax.ShapeDtypeStruct((NC, NPAD), jnp.float32),
    mesh=_SC_MESH,
    compiler_params=_SC_PARAMS,
    scratch_types=[
        pltpu.VMEM((ROWS_PER_TILE,), jnp.float32),
        pltpu.VMEM((BLK,), jnp.float32),
        pltpu.VMEM((NCHUNK, CPB, BLK), jnp.int32),
        pltpu.VMEM_SHARED((NPAD,), jnp.float32),
    ],
)

_fa_kernel = pl.kernel(
    _fa_body,
    out_type=jax.ShapeDtypeStruct((NC, NPAD, D), jnp.float32),
    mesh=_SC_MESH,
    compiler_params=_SC_PARAMS,
    scratch_types=[
        pltpu.VMEM((NPAD,), jnp.float32),
        pltpu.VMEM((NPAD,), jnp.float32),
        pltpu.VMEM((NPAD,), jnp.float32),
        pltpu.VMEM((CPB, BLK), jnp.int32),
        pltpu.VMEM((CPB, BLK), jnp.int32),
        pltpu.VMEM((BLK,), jnp.float32),
        pltpu.VMEM((BLK, D), jnp.float32),
        pltpu.SemaphoreType.DMA,
        pltpu.VMEM_SHARED((NPAD, D), jnp.float32),
    ],
)


def _tc_call(body, grid, in_specs, out_specs, out_shape):
    return pl.pallas_call(body, grid=grid, in_specs=in_specs,
                          out_specs=out_specs, out_shape=out_shape)


def kernel(x, edge_index, W1, b1, Wg0, bg0, Wg1, bg1, W2, b2):
    src = edge_index[0]
    dst = edge_index[1]
    pad_e = EPAD - E
    src3 = jnp.concatenate(
        [src, jnp.full((pad_e,), N, jnp.int32)]).reshape(
            NC * NS, NCHUNK, CPB, BLK)
    dst3 = jnp.concatenate(
        [dst, jnp.full((pad_e,), N, jnp.int32)]).reshape(
            NC * NS, NCHUNK, CPB, BLK)
    x_pad = jnp.zeros((NPAD, D), jnp.float32).at[:N].set(x)

    w1t = W1.T
    w2t = W2.T
    g0 = jnp.zeros((D, D), jnp.float32)
    g0 = g0.at[:, 0].set(Wg0[0, :D]).at[:, 1].set(Wg0[0, D:])
    g1 = jnp.zeros((D, D), jnp.float32)
    g1 = g1.at[:, 0].set(Wg1[0, :D]).at[:, 1].set(Wg1[0, D:])
    bgv0 = jnp.zeros((1, D), jnp.float32).at[0, 0].set(bg0[0])
    bgv1 = jnp.zeros((1, D), jnp.float32).at[0, 0].set(bg1[0])
    b1r = b1.reshape(1, D)
    b2r = b2.reshape(1, D)

    deg2 = _deg_kernel(dst3).reshape(NC, NPAD // D, D)

    RB = 2048
    GP = NPAD // RB
    SB = RB // D
    full = lambda shape: pl.BlockSpec(shape, lambda i: (0,) * len(shape))
    rows = pl.BlockSpec((RB, D), lambda i: (i, 0))

    h_pad, qr0, nrm2 = _tc_call(
        _prep_body, (GP,),
        [rows, full((D, D)), full((1, D)), full((D, D)), full((1, D)),
         pl.BlockSpec((NC, SB, D), lambda i: (0, i, 0))],
        [rows, rows, pl.BlockSpec((SB, D), lambda i: (i, 0))],
        [jax.ShapeDtypeStruct((NPAD, D), jnp.float32),
         jax.ShapeDtypeStruct((NPAD, D), jnp.float32),
         jax.ShapeDtypeStruct((NPAD // D, D), jnp.float32)],
    )(x_pad, w1t, b1r, g0, bgv0, deg2)

    nrm = nrm2.reshape(NPAD)
    agg0 = _fa_kernel(h_pad, qr0[:, 0], qr0[:, 1], nrm, src3, dst3)

    h1_pad, qr1 = _tc_call(
        _mid_body, (GP,),
        [rows, pl.BlockSpec((NC, RB, D), lambda i: (0, i, 0)),
         full((D, D)), full((1, D))],
        [rows, rows],
        [jax.ShapeDtypeStruct((NPAD, D), jnp.float32),
         jax.ShapeDtypeStruct((NPAD, D), jnp.float32)],
    )(h_pad, agg0, g1, bgv1)

    agg1 = _fa_kernel(h1_pad, qr1[:, 0], qr1[:, 1], nrm, src3, dst3)

    RB2 = 2000
    out = _tc_call(
        _final_body, (N // RB2,),
        [pl.BlockSpec((RB2, D), lambda i: (i, 0)),
         pl.BlockSpec((NC, RB2, D), lambda i: (0, i, 0)),
         full((D, D)), full((1, D))],
        pl.BlockSpec((RB2, D), lambda i: (i, 0)),
        jax.ShapeDtypeStruct((N, D), jnp.float32),
    )(h_pad, agg1, w2t, b2r)

    return out
